# baseline (speedup 1.0000x reference)
import functools

import jax
import jax.numpy as jnp
from jax import lax
from jax.experimental import pallas as pl
from jax.experimental.pallas import tpu as pltpu
from jax.experimental.pallas import tpu_sc as plsc

B = 16384
DIM = 16
ALL_DIM = 240
USER_NUM = 1000000
ITEM_NUM = 100000
NW = 32
CHUNK = 128
ROWS_PER_W = B // NW
BLK = 512
N_BLK = B // BLK
G_PER_W = 2 * DIM * (ROWS_PER_W // CHUNK)
EBUF = 2 * DIM * ROWS_PER_W

_WIDTHS = [8, 3, 21] + [19] * 10
_OFFS = []
_o = 0
for _w in _WIDTHS:
    _OFFS.append(_o)
    _o += _w
OH_DIM = _o
OH_PAD = 224
SMALL_DIM = 16 * 13


def _sc_gather(tu_flat, ti_flat, idx_all):
    mesh = plsc.VectorSubcoreMesh(core_axis_name="c", subcore_axis_name="s")

    @functools.partial(
        pl.kernel,
        out_type=jax.ShapeDtypeStruct((NW * EBUF,), jnp.float32),
        mesh=mesh,
        scratch_types=[
            pltpu.VMEM((G_PER_W, CHUNK), jnp.int32),
            pltpu.VMEM((EBUF,), jnp.float32),
            pltpu.SemaphoreType.DMA,
        ],
    )
    def gather_k(tu, ti, idx_hbm, out_hbm, idx_v, ebuf, sem):
        wid = lax.axis_index("s") * 2 + lax.axis_index("c")
        pltpu.sync_copy(idx_hbm.at[pl.ds(wid * G_PER_W, G_PER_W)], idx_v)
        for t, tbl in ((0, tu), (1, ti)):
            for batch in range(4):
                g0 = t * 64 + batch * 16

                def fire(i, _, tbl=tbl, g0=g0):
                    g = g0 + i
                    pltpu.async_copy(tbl.at[idx_v.at[g]],
                                     ebuf.at[pl.ds(g * CHUNK, CHUNK)], sem)
                    return 0

                lax.fori_loop(0, 16, fire, 0)
                pltpu.make_async_copy(
                    tbl.at[pl.ds(0, 16 * CHUNK)],
                    ebuf.at[pl.ds(g0 * CHUNK, 16 * CHUNK)], sem).wait()
        pltpu.sync_copy(ebuf, out_hbm.at[pl.ds(wid * EBUF, EBUF)])

    return gather_k(tu_flat, ti_flat, idx_all)


def _tc_forward_body(ut_ref, it_ref, idx_ref, y_ref, bd_ref,
                     W1, b1, W2, b2, W3, b3,
                     M1, c1, M2, c2, M3, c3, M4t, c4,
                     s_ref, sb_ref, loss_ref):
    colid = lax.broadcasted_iota(jnp.int32, (BLK, OH_PAD), 1)
    oh = jnp.zeros((BLK, OH_PAD), jnp.float32)
    for g in range(13):
        idxg = idx_ref[:, g:g + 1] + _OFFS[g]
        oh = oh + jnp.where(colid == idxg, 1.0, 0.0)
    small = jnp.dot(oh, bd_ref[...], preferred_element_type=jnp.float32)

    uw = jnp.transpose(ut_ref[...], (1, 0))
    iw = jnp.transpose(it_ref[...], (1, 0))

    x0 = jnp.concatenate([uw, iw, small], axis=1)
    xl = x0
    for W, bb in ((W1, b1), (W2, b2), (W3, b3)):
        z = jnp.dot(xl, W[...], preferred_element_type=jnp.float32) + bb[...][None, :]
        xl = x0 * z + xl
    h = jnp.maximum(jnp.dot(xl, M1[...], preferred_element_type=jnp.float32)
                    + c1[...][None, :], 0.0)
    h = jnp.maximum(jnp.dot(h, M2[...], preferred_element_type=jnp.float32)
                    + c2[...][None, :], 0.0)
    h = jnp.maximum(jnp.dot(h, M3[...], preferred_element_type=jnp.float32)
                    + c3[...][None, :], 0.0)
    logit = jnp.sum(h * M4t[...], axis=1, keepdims=True) + c4[...][None, :]
    sig = 1.0 / (1.0 + jnp.exp(-logit))
    s_ref[...] = sig
    sb_ref[...] = jnp.broadcast_to(sig, (BLK, 128))
    y = y_ref[...]
    terms = y * jnp.log(sig + 1e-6) + (1.0 - y) * jnp.log(1.0 - sig + 1e-6)
    part = -jnp.sum(terms)

    @pl.when(pl.program_id(0) == 0)
    def _():
        loss_ref[...] = jnp.zeros_like(loss_ref)

    loss_ref[...] += jnp.full((1, 128), part, jnp.float32)


def _tc_forward(ut, it, idx_small, y, bd,
                W1, b1, W2, b2, W3, b3, M1, c1, M2, c2, M3, c3, M4t, c4):
    full = lambda *shape: pl.BlockSpec(shape, lambda i: tuple(0 for _ in shape))
    return pl.pallas_call(
        _tc_forward_body,
        grid=(N_BLK,),
        in_specs=[
            pl.BlockSpec((DIM, BLK), lambda i: (0, i)),
            pl.BlockSpec((DIM, BLK), lambda i: (0, i)),
            pl.BlockSpec((BLK, 16), lambda i: (i, 0)),
            pl.BlockSpec((BLK, 1), lambda i: (i, 0)),
            full(OH_PAD, SMALL_DIM),
            full(ALL_DIM, ALL_DIM), full(ALL_DIM),
            full(ALL_DIM, ALL_DIM), full(ALL_DIM),
            full(ALL_DIM, ALL_DIM), full(ALL_DIM),
            full(ALL_DIM, 128), full(128),
            full(128, 64), full(64),
            full(64, 32), full(32),
            full(1, 32), full(1),
        ],
        out_specs=[
            pl.BlockSpec((BLK, 1), lambda i: (i, 0)),
            pl.BlockSpec((BLK, 128), lambda i: (i, 0)),
            pl.BlockSpec((1, 128), lambda i: (0, 0)),
        ],
        out_shape=[
            jax.ShapeDtypeStruct((B, 1), jnp.float32),
            jax.ShapeDtypeStruct((B, 128), jnp.float32),
            jax.ShapeDtypeStruct((1, 128), jnp.float32),
        ],
        compiler_params=pltpu.CompilerParams(
            dimension_semantics=("arbitrary",)),
    )(ut, it, idx_small, y, bd, W1, b1, W2, b2, W3, b3,
      M1, c1, M2, c2, M3, c3, M4t, c4)


def _auc_body(si_ref, yp_ref, sb_ref, num_ref, npos_ref):
    si4 = si_ref[...]
    yp4 = yp_ref[...]
    part = jnp.float32(0.0)
    for pair in range(4):
        si_a = jnp.broadcast_to(si4[2 * pair:2 * pair + 1, :], (8, 128))
        si_b = jnp.broadcast_to(si4[2 * pair + 1:2 * pair + 2, :], (8, 128))

        def jb(jc, accs, si_a=si_a, si_b=si_b):
            acc_a, acc_b = accs
            jt = sb_ref[pl.ds(jc * 8, 8), :]
            acc_a = acc_a + jnp.where(jt < si_a, 1.0, 0.0)
            acc_b = acc_b + jnp.where(jt < si_b, 1.0, 0.0)
            return (acc_a, acc_b)

        acc_a, acc_b = lax.fori_loop(
            0, B // 8, jb,
            (jnp.zeros((8, 128), jnp.float32), jnp.zeros((8, 128), jnp.float32)),
            unroll=8)
        cnt_a = jnp.sum(acc_a, axis=0, keepdims=True)
        cnt_b = jnp.sum(acc_b, axis=0, keepdims=True)
        part += jnp.sum(yp4[2 * pair:2 * pair + 1, :] * cnt_a)
        part += jnp.sum(yp4[2 * pair + 1:2 * pair + 2, :] * cnt_b)
    pospart = jnp.sum(yp4)

    @pl.when(pl.program_id(0) == 0)
    def _():
        num_ref[...] = jnp.zeros_like(num_ref)
        npos_ref[...] = jnp.zeros_like(npos_ref)

    num_ref[...] += jnp.full((1, 128), part, jnp.float32)
    npos_ref[...] += jnp.full((1, 128), pospart, jnp.float32)


def _auc_kernel(s_mat, ypos_mat, s_bcast):
    return pl.pallas_call(
        _auc_body,
        grid=(B // 1024,),
        in_specs=[
            pl.BlockSpec((8, 128), lambda i: (i, 0)),
            pl.BlockSpec((8, 128), lambda i: (i, 0)),
            pl.BlockSpec((B, 128), lambda i: (0, 0)),
        ],
        out_specs=[
            pl.BlockSpec((1, 128), lambda i: (0, 0)),
            pl.BlockSpec((1, 128), lambda i: (0, 0)),
        ],
        out_shape=[
            jax.ShapeDtypeStruct((1, 128), jnp.float32),
            jax.ShapeDtypeStruct((1, 128), jnp.float32),
        ],
        compiler_params=pltpu.CompilerParams(
            dimension_semantics=("arbitrary",)),
    )(s_mat, ypos_mat, s_bcast)


def kernel(userid, itemid, user_age, gender, user_occupation, item_kind, label,
           emb_user, emb_item, emb_age, emb_gender, emb_occ, emb_kind,
           W_dcn1, b_dcn1, W_dcn2, b_dcn2, W_dcn3, b_dcn3,
           W_mlp1, b_mlp1, W_mlp2, b_mlp2, W_mlp3, b_mlp3, W_mlp4, b_mlp4):
    uid = userid[:, 0].astype(jnp.int32)
    tid = itemid[:, 0].astype(jnp.int32)

    f16 = jnp.arange(DIM, dtype=jnp.int32)
    au = (uid.reshape(1, NW, 4, CHUNK)
          + (f16 * USER_NUM).reshape(DIM, 1, 1, 1))
    ai = (tid.reshape(1, NW, 4, CHUNK)
          + (f16 * ITEM_NUM).reshape(DIM, 1, 1, 1))
    idx_all = (jnp.stack([au.transpose(1, 0, 2, 3),
                          ai.transpose(1, 0, 2, 3)], axis=1)
               .reshape(NW * G_PER_W, CHUNK))

    g_flat = _sc_gather(emb_user.T.reshape(-1), emb_item.T.reshape(-1), idx_all)
    g = g_flat.reshape(NW, 2, DIM, ROWS_PER_W).transpose(1, 2, 0, 3)
    g = g.reshape(2, DIM, B)
    ut, it = g[0], g[1]

    idx_small = jnp.concatenate(
        [user_age.astype(jnp.int32), gender.astype(jnp.int32),
         user_occupation.astype(jnp.int32), item_kind.astype(jnp.int32),
         jnp.zeros((B, 3), jnp.int32)], axis=1)

    emb_kind_z = emb_kind.at[0].set(0.0)
    bd = jnp.zeros((OH_PAD, SMALL_DIM), jnp.float32)
    small_tabs = [emb_age, emb_gender, emb_occ] + [emb_kind_z] * 10
    for g_ in range(13):
        bd = lax.dynamic_update_slice(bd, small_tabs[g_], (_OFFS[g_], 16 * g_))

    y = label.astype(jnp.float32)
    s_col, s_bcast, loss_vec = _tc_forward(
        ut, it, idx_small, y, bd,
        W_dcn1, b_dcn1, W_dcn2, b_dcn2, W_dcn3, b_dcn3,
        W_mlp1, b_mlp1, W_mlp2, b_mlp2, W_mlp3, b_mlp3,
        W_mlp4.T, b_mlp4)

    s_mat = s_col.reshape(B // 128, 128)
    ypos_mat = y.reshape(B // 128, 128)
    num_vec, npos_vec = _auc_kernel(s_mat, ypos_mat, s_bcast)

    loss = loss_vec[0, 0] / jnp.float32(B)
    npos = npos_vec[0, 0]
    nneg = jnp.float32(B) - npos
    numerator = num_vec[0, 0] - npos * (npos - 1.0) / 2.0
    auc = numerator / (npos * nneg)
    return (loss, auc)


# speedup vs baseline: 1.9891x; 1.9891x over previous
import functools

import jax
import jax.numpy as jnp
from jax import lax
from jax.experimental import pallas as pl
from jax.experimental.pallas import tpu as pltpu
from jax.experimental.pallas import tpu_sc as plsc

B = 16384
DIM = 16
ALL_DIM = 240
USER_NUM = 1000000
ITEM_NUM = 100000
NW = 32
CHUNK = 128
ROWS_PER_W = B // NW
N_CHUNK = ROWS_PER_W // CHUNK
BLK = 512
N_BLK = B // BLK

_WIDTHS = [8, 3, 21] + [19] * 10
_OFFS = []
_o = 0
for _w in _WIDTHS:
    _OFFS.append(_o)
    _o += _w
OH_DIM = _o
OH_PAD = 224
SMALL_DIM = 16 * 13


def _sc_gather(t_user, t_item, idx_all):
    mesh = plsc.VectorSubcoreMesh(core_axis_name="c", subcore_axis_name="s")

    @functools.partial(
        pl.kernel,
        out_type=[
            jax.ShapeDtypeStruct((B, DIM), jnp.float32),
            jax.ShapeDtypeStruct((B, DIM), jnp.float32),
        ],
        mesh=mesh,
        scratch_types=[
            pltpu.VMEM((2, N_CHUNK, CHUNK), jnp.int32),
            pltpu.VMEM((2, ROWS_PER_W, DIM), jnp.float32),
            pltpu.SemaphoreType.DMA,
        ],
        compiler_params=pltpu.CompilerParams(use_tc_tiling_on_sc=False),
    )
    def gather_k(tu, ti, idx_hbm, out_u, out_i, idx_v, rbuf, sem):
        wid = lax.axis_index("s") * 2 + lax.axis_index("c")
        pltpu.sync_copy(idx_hbm.at[wid], idx_v)
        copies = []
        for slot, table in ((0, tu), (1, ti)):
            for c in range(N_CHUNK):
                copies.append(pltpu.async_copy(
                    table.at[idx_v.at[slot, c]],
                    rbuf.at[slot, pl.ds(c * CHUNK, CHUNK)], sem))
        for cp in copies:
            cp.wait()
        base = wid * ROWS_PER_W
        pltpu.sync_copy(rbuf.at[0], out_u.at[pl.ds(base, ROWS_PER_W)])
        pltpu.sync_copy(rbuf.at[1], out_i.at[pl.ds(base, ROWS_PER_W)])

    return gather_k(t_user, t_item, idx_all)


def _tc_forward_body(uw_ref, iw_ref, idx_ref, y_ref, bd_ref,
                     W1, b1, W2, b2, W3, b3,
                     M1, c1, M2, c2, M3, c3, M4t, c4,
                     s_ref, sb_ref, loss_ref):
    colid = lax.broadcasted_iota(jnp.int32, (BLK, OH_PAD), 1)
    oh = jnp.zeros((BLK, OH_PAD), jnp.float32)
    for g in range(13):
        idxg = idx_ref[:, g:g + 1] + _OFFS[g]
        oh = oh + jnp.where(colid == idxg, 1.0, 0.0)
    small = jnp.dot(oh, bd_ref[...], preferred_element_type=jnp.float32)

    uw = uw_ref[...]
    iw = iw_ref[...]

    x0 = jnp.concatenate([uw, iw, small], axis=1)
    xl = x0
    for W, bb in ((W1, b1), (W2, b2), (W3, b3)):
        z = jnp.dot(xl, W[...], preferred_element_type=jnp.float32) + bb[...][None, :]
        xl = x0 * z + xl
    h = jnp.maximum(jnp.dot(xl, M1[...], preferred_element_type=jnp.float32)
                    + c1[...][None, :], 0.0)
    h = jnp.maximum(jnp.dot(h, M2[...], preferred_element_type=jnp.float32)
                    + c2[...][None, :], 0.0)
    h = jnp.maximum(jnp.dot(h, M3[...], preferred_element_type=jnp.float32)
                    + c3[...][None, :], 0.0)
    logit = jnp.sum(h * M4t[...], axis=1, keepdims=True) + c4[...][None, :]
    sig = 1.0 / (1.0 + jnp.exp(-logit))
    s_ref[...] = sig
    sb_ref[...] = jnp.broadcast_to(sig, (BLK, 128))
    y = y_ref[...]
    terms = y * jnp.log(sig + 1e-6) + (1.0 - y) * jnp.log(1.0 - sig + 1e-6)
    part = -jnp.sum(terms)

    @pl.when(pl.program_id(0) == 0)
    def _():
        loss_ref[...] = jnp.zeros_like(loss_ref)

    loss_ref[...] += jnp.full((1, 128), part, jnp.float32)


def _tc_forward(uw, iw, idx_small, y, bd,
                W1, b1, W2, b2, W3, b3, M1, c1, M2, c2, M3, c3, M4t, c4):
    full = lambda *shape: pl.BlockSpec(shape, lambda i: tuple(0 for _ in shape))
    return pl.pallas_call(
        _tc_forward_body,
        grid=(N_BLK,),
        in_specs=[
            pl.BlockSpec((BLK, DIM), lambda i: (i, 0)),
            pl.BlockSpec((BLK, DIM), lambda i: (i, 0)),
            pl.BlockSpec((BLK, 16), lambda i: (i, 0)),
            pl.BlockSpec((BLK, 1), lambda i: (i, 0)),
            full(OH_PAD, SMALL_DIM),
            full(ALL_DIM, ALL_DIM), full(ALL_DIM),
            full(ALL_DIM, ALL_DIM), full(ALL_DIM),
            full(ALL_DIM, ALL_DIM), full(ALL_DIM),
            full(ALL_DIM, 128), full(128),
            full(128, 64), full(64),
            full(64, 32), full(32),
            full(1, 32), full(1),
        ],
        out_specs=[
            pl.BlockSpec((BLK, 1), lambda i: (i, 0)),
            pl.BlockSpec((BLK, 128), lambda i: (i, 0)),
            pl.BlockSpec((1, 128), lambda i: (0, 0)),
        ],
        out_shape=[
            jax.ShapeDtypeStruct((B, 1), jnp.float32),
            jax.ShapeDtypeStruct((B, 128), jnp.float32),
            jax.ShapeDtypeStruct((1, 128), jnp.float32),
        ],
        compiler_params=pltpu.CompilerParams(
            dimension_semantics=("arbitrary",)),
    )(uw, iw, idx_small, y, bd, W1, b1, W2, b2, W3, b3,
      M1, c1, M2, c2, M3, c3, M4t, c4)


def _auc_body(si_ref, yp_ref, sb_ref, num_ref, npos_ref):
    si4 = si_ref[...]
    yp4 = yp_ref[...]
    part = jnp.float32(0.0)
    for pair in range(4):
        si_a = jnp.broadcast_to(si4[2 * pair:2 * pair + 1, :], (8, 128))
        si_b = jnp.broadcast_to(si4[2 * pair + 1:2 * pair + 2, :], (8, 128))

        def jb(jc, accs, si_a=si_a, si_b=si_b):
            acc_a, acc_b = accs
            jt = sb_ref[pl.ds(jc * 8, 8), :]
            acc_a = acc_a + jnp.where(jt < si_a, 1.0, 0.0)
            acc_b = acc_b + jnp.where(jt < si_b, 1.0, 0.0)
            return (acc_a, acc_b)

        acc_a, acc_b = lax.fori_loop(
            0, B // 8, jb,
            (jnp.zeros((8, 128), jnp.float32), jnp.zeros((8, 128), jnp.float32)),
            unroll=8)
        cnt_a = jnp.sum(acc_a, axis=0, keepdims=True)
        cnt_b = jnp.sum(acc_b, axis=0, keepdims=True)
        part += jnp.sum(yp4[2 * pair:2 * pair + 1, :] * cnt_a)
        part += jnp.sum(yp4[2 * pair + 1:2 * pair + 2, :] * cnt_b)
    pospart = jnp.sum(yp4)

    @pl.when(pl.program_id(0) == 0)
    def _():
        num_ref[...] = jnp.zeros_like(num_ref)
        npos_ref[...] = jnp.zeros_like(npos_ref)

    num_ref[...] += jnp.full((1, 128), part, jnp.float32)
    npos_ref[...] += jnp.full((1, 128), pospart, jnp.float32)


def _auc_kernel(s_mat, ypos_mat, s_bcast):
    return pl.pallas_call(
        _auc_body,
        grid=(B // 1024,),
        in_specs=[
            pl.BlockSpec((8, 128), lambda i: (i, 0)),
            pl.BlockSpec((8, 128), lambda i: (i, 0)),
            pl.BlockSpec((B, 128), lambda i: (0, 0)),
        ],
        out_specs=[
            pl.BlockSpec((1, 128), lambda i: (0, 0)),
            pl.BlockSpec((1, 128), lambda i: (0, 0)),
        ],
        out_shape=[
            jax.ShapeDtypeStruct((1, 128), jnp.float32),
            jax.ShapeDtypeStruct((1, 128), jnp.float32),
        ],
        compiler_params=pltpu.CompilerParams(
            dimension_semantics=("arbitrary",)),
    )(s_mat, ypos_mat, s_bcast)


def kernel(userid, itemid, user_age, gender, user_occupation, item_kind, label,
           emb_user, emb_item, emb_age, emb_gender, emb_occ, emb_kind,
           W_dcn1, b_dcn1, W_dcn2, b_dcn2, W_dcn3, b_dcn3,
           W_mlp1, b_mlp1, W_mlp2, b_mlp2, W_mlp3, b_mlp3, W_mlp4, b_mlp4):
    ui = jnp.concatenate([userid, itemid], axis=1).astype(jnp.int32)
    idx_all = ui.T.reshape(2, NW, N_CHUNK, CHUNK).transpose(1, 0, 2, 3)

    uw, iw = _sc_gather(emb_user, emb_item, idx_all)

    idx_small = jnp.concatenate(
        [user_age.astype(jnp.int32), gender.astype(jnp.int32),
         user_occupation.astype(jnp.int32), item_kind.astype(jnp.int32),
         jnp.zeros((B, 3), jnp.int32)], axis=1)

    emb_kind_z = emb_kind.at[0].set(0.0)
    bd = jnp.zeros((OH_PAD, SMALL_DIM), jnp.float32)
    small_tabs = [emb_age, emb_gender, emb_occ] + [emb_kind_z] * 10
    for g in range(13):
        bd = lax.dynamic_update_slice(bd, small_tabs[g], (_OFFS[g], 16 * g))

    y = label.astype(jnp.float32)
    s_col, s_bcast, loss_vec = _tc_forward(
        uw, iw, idx_small, y, bd,
        W_dcn1, b_dcn1, W_dcn2, b_dcn2, W_dcn3, b_dcn3,
        W_mlp1, b_mlp1, W_mlp2, b_mlp2, W_mlp3, b_mlp3,
        W_mlp4.T, b_mlp4)

    s_mat = s_col.reshape(B // 128, 128)
    ypos_mat = y.reshape(B // 128, 128)
    num_vec, npos_vec = _auc_kernel(s_mat, ypos_mat, s_bcast)

    loss = loss_vec[0, 0] / jnp.float32(B)
    npos = npos_vec[0, 0]
    nneg = jnp.float32(B) - npos
    numerator = num_vec[0, 0] - npos * (npos - 1.0) / 2.0
    auc = numerator / (npos * nneg)
    return (loss, auc)
